# Initial kernel scaffold; baseline (speedup 1.0000x reference)
import jax
import jax.numpy as jnp
from jax.experimental import pallas as pl


def kernel(x, hilbert_perm):
    raise NotImplementedError("write your pallas kernel here")


# speedup vs baseline: 12.1930x; 12.1930x over previous
import jax
import jax.numpy as jnp
from jax.experimental import pallas as pl


def _scale2_kernel(x_ref, o_ref):
    o_ref[...] = x_ref[...] * 2.0


def kernel(x, hilbert_perm):
    del hilbert_perm
    B, N, C = x.shape
    xf = x.reshape(B * N, C)
    blk = 1024
    out = pl.pallas_call(
        _scale2_kernel,
        grid=(pl.cdiv(B * N, blk),),
        in_specs=[pl.BlockSpec((blk, C), lambda i: (i, 0))],
        out_specs=pl.BlockSpec((blk, C), lambda i: (i, 0)),
        out_shape=jax.ShapeDtypeStruct((B * N, C), x.dtype),
    )(xf)
    return out.reshape(B, N, C)


# speedup vs baseline: 13.1111x; 1.0753x over previous
import jax
import jax.numpy as jnp
from jax.experimental import pallas as pl


def _scale2_kernel(x_ref, o_ref):
    o_ref[...] = x_ref[...] * 2.0


def kernel(x, hilbert_perm):
    del hilbert_perm
    B, N, C = x.shape
    xf = x.reshape(B * N, C)
    blk = 2048
    out = pl.pallas_call(
        _scale2_kernel,
        grid=(pl.cdiv(B * N, blk),),
        in_specs=[pl.BlockSpec((blk, C), lambda i: (i, 0))],
        out_specs=pl.BlockSpec((blk, C), lambda i: (i, 0)),
        out_shape=jax.ShapeDtypeStruct((B * N, C), x.dtype),
    )(xf)
    return out.reshape(B, N, C)
